# baseline (speedup 1.0000x reference)
import functools
import math

import jax
import jax.numpy as jnp
from jax import lax
from jax.experimental import pallas as pl
from jax.experimental.pallas import tpu as pltpu
from jax.experimental.pallas import tpu_sc as plsc

NC, NS, LANES = 2, 16, 16
NW = NC * NS


def _mesh():
    return plsc.VectorSubcoreMesh(
        core_axis_name="c", subcore_axis_name="s",
        num_cores=NC, num_subcores=NS)


def _make_hist(NP, CH):
    ZCH = NP // NS

    @functools.partial(
        pl.kernel,
        out_type=jax.ShapeDtypeStruct((NC * NP,), jnp.float32),
        mesh=_mesh(),
        scratch_types=[
            pltpu.VMEM((CH, 128), jnp.int32),
            pltpu.VMEM((128,), jnp.float32),
            pltpu.VMEM((ZCH,), jnp.float32),
            pltpu.VMEM_SHARED((NP,), jnp.float32),
            pltpu.SemaphoreType.DMA,
        ],
    )
    def hist(dst_hbm, out_hbm, didx, ones_v, zb, acc, sem):
        c = lax.axis_index("c")
        s = lax.axis_index("s")
        t = c * NS + s

        def fill_z(i, carry):
            zb[pl.ds(i * 16, 16)] = jnp.zeros((16,), jnp.float32)
            return carry
        lax.fori_loop(0, ZCH // 16, fill_z, 0)

        def fill_o(i, carry):
            ones_v[pl.ds(i * 16, 16)] = jnp.full((16,), 1.0, jnp.float32)
            return carry
        lax.fori_loop(0, 128 // 16, fill_o, 0)

        pltpu.sync_copy(zb, acc.at[pl.ds(s * ZCH, ZCH)])
        plsc.subcore_barrier()

        pltpu.sync_copy(dst_hbm.at[pl.ds(t * CH, CH)], didx)

        def fire(j, carry):
            pltpu.async_copy(ones_v, acc.at[didx.at[j]], sem, add=True)
            return carry
        lax.fori_loop(0, CH, fire, 0)

        def drain(j, carry):
            pltpu.make_async_copy(ones_v, acc.at[didx.at[0]], sem).wait()
            return carry
        lax.fori_loop(0, CH, drain, 0)

        plsc.subcore_barrier()
        pltpu.sync_copy(acc.at[pl.ds(s * ZCH, ZCH)], zb)
        pltpu.sync_copy(zb, out_hbm.at[pl.ds(c * NP + s * ZCH, ZCH)])

    return hist


def _make_segsum(NR, R, H, A):
    RPT = NR // NS
    PH = 8
    TOT = R // (NS * PH)
    B = TOT - A
    assert 0 < A < TOT

    @functools.partial(
        pl.kernel,
        out_type=jax.ShapeDtypeStruct((NC, NR, H), jnp.float32),
        mesh=_mesh(),
        scratch_types=[
            pltpu.VMEM((PH, 128), jnp.int32),
            pltpu.VMEM((PH, 128), jnp.int32),
            pltpu.VMEM((128, H), jnp.float32),
            pltpu.VMEM((128, H), jnp.float32),
            pltpu.VMEM_SHARED((NR, H), jnp.float32),
            pltpu.SemaphoreType.DMA,
            pltpu.SemaphoreType.DMA,
            pltpu.SemaphoreType.DMA,
            pltpu.SemaphoreType.DMA,
        ],
    )
    def segsum(g_hbm, src_hbm, dst_hbm, out_hbm,
               sidx, didx, buf0, buf1, acc, gs0, gs1, ss0, ss1):
        c = lax.axis_index("c")
        s = lax.axis_index("s")

        def zrow(i, carry):
            r = i // (H // 16)
            k = i % (H // 16)
            buf0[r, pl.ds(k * 16, 16)] = jnp.zeros((16,), jnp.float32)
            return carry
        lax.fori_loop(0, 128 * (H // 16), zrow, 0)

        base = s * RPT
        off = 0
        rem = RPT
        while rem > 0:
            sz = min(128, rem)
            pltpu.sync_copy(buf0.at[pl.ds(0, sz)],
                            acc.at[pl.ds(base + off, sz)])
            off += sz
            rem -= sz
        plsc.subcore_barrier()

        def run_phases(nph, row0):
            def phase_body(ph, carry):
                row = row0 + ph * PH
                pltpu.sync_copy(src_hbm.at[pl.ds(row, PH)], sidx)
                pltpu.sync_copy(dst_hbm.at[pl.ds(row, PH)], didx)
                pltpu.async_copy(g_hbm.at[sidx.at[0]], buf0, gs0)

                def pair(p, carry2):
                    j0 = 2 * p
                    j1 = j0 + 1
                    pltpu.make_async_copy(
                        g_hbm.at[sidx.at[j0]], buf0, gs0).wait()
                    pltpu.async_copy(buf0, acc.at[didx.at[j0]], ss0,
                                     add=True)

                    @pl.when(p > 0)
                    def _():
                        pltpu.make_async_copy(
                            buf1, acc.at[didx.at[j1]], ss1).wait()

                    pltpu.async_copy(g_hbm.at[sidx.at[j1]], buf1, gs1)
                    pltpu.make_async_copy(
                        g_hbm.at[sidx.at[j1]], buf1, gs1).wait()
                    pltpu.async_copy(buf1, acc.at[didx.at[j1]], ss1,
                                     add=True)
                    pltpu.make_async_copy(
                        buf0, acc.at[didx.at[j0]], ss0).wait()

                    @pl.when(j1 + 1 < PH)
                    def _():
                        pltpu.async_copy(g_hbm.at[sidx.at[j1 + 1]], buf0, gs0)

                    return carry2
                lax.fori_loop(0, PH // 2, pair, 0)
                pltpu.make_async_copy(
                    buf1, acc.at[didx.at[PH - 1]], ss1).wait()
                return carry
            lax.fori_loop(0, nph, phase_body, 0)

        @pl.when(c == 0)
        def _():
            run_phases(A, s * (A * PH))

        @pl.when(c == 1)
        def _():
            run_phases(B, NS * A * PH + s * (B * PH))

        plsc.subcore_barrier()
        off = 0
        rem = RPT
        while rem > 0:
            sz = min(128, rem)
            pltpu.sync_copy(acc.at[pl.ds(base + off, sz)],
                            buf0.at[pl.ds(0, sz)])
            pltpu.sync_copy(buf0.at[pl.ds(0, sz)],
                            out_hbm.at[c, pl.ds(base + off, sz)])
            off += sz
            rem -= sz

    return segsum


def _dinv_col(degp, N):
    NP = degp.shape[1]
    rows = NP // 128

    def body(d_ref, o_ref):
        d = d_ref[0] + d_ref[1] + 1.0
        o_ref[...] = lax.rsqrt(d)

    dinv2d = pl.pallas_call(
        body,
        out_shape=jax.ShapeDtypeStruct((rows, 128), jnp.float32),
    )(degp.reshape(NC, rows, 128))
    return dinv2d.reshape(NP)[:N, None]


def _tc_in(x, W, dinv_col, B=400):
    N, D = x.shape
    H = W.shape[1]

    def body(x_ref, w_ref, dv_ref, o_ref):
        o_ref[...] = dv_ref[...] * jnp.dot(
            x_ref[...], w_ref[...], preferred_element_type=jnp.float32)

    return pl.pallas_call(
        body, grid=(N // B,),
        in_specs=[pl.BlockSpec((B, D), lambda i: (i, 0)),
                  pl.BlockSpec((D, H), lambda i: (0, 0)),
                  pl.BlockSpec((B, 1), lambda i: (i, 0))],
        out_specs=pl.BlockSpec((B, H), lambda i: (i, 0)),
        out_shape=jax.ShapeDtypeStruct((N, H), jnp.float32),
    )(x, W, dinv_col)


def _tc_mid(s1p, g1, dinv_col, b1, gamma1, beta1, W2, B=400):
    N, H = g1.shape
    NR = s1p.shape[1]
    bnscale = 1.0 / math.sqrt(1.0 + 1e-5)

    def body(s_ref, g_ref, dv_ref, b_ref, ga_ref, be_ref, w_ref,
             x1_ref, g2_ref):
        dv = dv_ref[...]
        pre = dv * (s_ref[0] + s_ref[1] + g_ref[...]) + b_ref[...]
        x1 = jnp.maximum(pre * (ga_ref[...] * bnscale) + be_ref[...], 0.0)
        x1_ref[...] = x1
        g2_ref[...] = dv * jnp.dot(
            x1, w_ref[...], preferred_element_type=jnp.float32)

    return pl.pallas_call(
        body, grid=(N // B,),
        in_specs=[pl.BlockSpec((NC, B, H), lambda i: (0, i, 0)),
                  pl.BlockSpec((B, H), lambda i: (i, 0)),
                  pl.BlockSpec((B, 1), lambda i: (i, 0)),
                  pl.BlockSpec((1, H), lambda i: (0, 0)),
                  pl.BlockSpec((1, H), lambda i: (0, 0)),
                  pl.BlockSpec((1, H), lambda i: (0, 0)),
                  pl.BlockSpec((H, H), lambda i: (0, 0))],
        out_specs=[pl.BlockSpec((B, H), lambda i: (i, 0)),
                   pl.BlockSpec((B, H), lambda i: (i, 0))],
        out_shape=[jax.ShapeDtypeStruct((N, H), jnp.float32),
                   jax.ShapeDtypeStruct((N, H), jnp.float32)],
    )(s1p[:, :N], g1, dinv_col, b1, gamma1, beta1, W2)


def _tc_out(s2p, g2, x1, dinv_col, b2, Wf, bf, B=400):
    N, H = g2.shape
    C = Wf.shape[1]

    def body(s_ref, g_ref, x1_ref, dv_ref, b_ref, wf_ref, bf_ref, o_ref):
        x2 = dv_ref[...] * (s_ref[0] + s_ref[1] + g_ref[...]) + b_ref[...]
        xj = jnp.maximum(x1_ref[...], x2)
        o = jnp.dot(xj, wf_ref[...],
                    preferred_element_type=jnp.float32) + bf_ref[...]
        m = jnp.max(o, axis=1, keepdims=True)
        e = jnp.exp(o - m)
        o_ref[...] = (o - m) - jnp.log(jnp.sum(e, axis=1, keepdims=True))

    return pl.pallas_call(
        body, grid=(N // B,),
        in_specs=[pl.BlockSpec((NC, B, H), lambda i: (0, i, 0)),
                  pl.BlockSpec((B, H), lambda i: (i, 0)),
                  pl.BlockSpec((B, H), lambda i: (i, 0)),
                  pl.BlockSpec((B, 1), lambda i: (i, 0)),
                  pl.BlockSpec((1, H), lambda i: (0, 0)),
                  pl.BlockSpec((H, C), lambda i: (0, 0)),
                  pl.BlockSpec((1, C), lambda i: (0, 0))],
        out_specs=pl.BlockSpec((B, C), lambda i: (i, 0)),
        out_shape=jax.ShapeDtypeStruct((N, C), jnp.float32),
    )(s2p[:, :N], g2, x1, dinv_col, b2, Wf, bf)


def kernel(x, edge_index, W1, b1, gamma1, beta1, W2, b2, Wf, bf):
    N, D = x.shape
    H = W1.shape[1]
    E = edge_index.shape[1]

    EP = ((E + NW * 2048 - 1) // (NW * 2048)) * (NW * 2048)
    CH = EP // (NW * 128)
    pad = EP - E
    src = edge_index[0].astype(jnp.int32)
    dst = edge_index[1].astype(jnp.int32)
    src2d = jnp.concatenate([src, jnp.zeros((pad,), jnp.int32)]).reshape(-1, 128)
    dst2d = jnp.concatenate([dst, jnp.full((pad,), N, jnp.int32)]).reshape(-1, 128)

    NP = ((N + 1 + 127) // 128) * 128
    NR = NP

    degp = _make_hist(NP, CH)(dst2d).reshape(NC, NP)
    dinv_col = _dinv_col(degp, N)

    R = EP // 128
    TOT = R // (NS * 8)
    A = max(1, min(TOT - 1, (TOT * 17 + 10) // 20))
    g1 = _tc_in(x, W1, dinv_col)
    s1p = _make_segsum(NR, R, H, A)(g1, src2d, dst2d)
    x1, g2 = _tc_mid(s1p, g1, dinv_col,
                     b1[None, :], gamma1[None, :], beta1[None, :], W2)
    s2p = _make_segsum(NR, R, H, A)(g2, src2d, dst2d)
    return _tc_out(s2p, g2, x1, dinv_col, b2[None, :], Wf, bf[None, :])


# speedup vs baseline: 1.1016x; 1.1016x over previous
import functools
import math

import jax
import jax.numpy as jnp
from jax import lax
from jax.experimental import pallas as pl
from jax.experimental.pallas import tpu as pltpu
from jax.experimental.pallas import tpu_sc as plsc

NC, NS, LANES = 2, 16, 16
NW = NC * NS


def _mesh():
    return plsc.VectorSubcoreMesh(
        core_axis_name="c", subcore_axis_name="s",
        num_cores=NC, num_subcores=NS)


def _make_hist(NP, CH):
    ZCH = NP // NS

    @functools.partial(
        pl.kernel,
        out_type=jax.ShapeDtypeStruct((NC * NP,), jnp.float32),
        mesh=_mesh(),
        scratch_types=[
            pltpu.VMEM((CH, 128), jnp.int32),
            pltpu.VMEM((128,), jnp.float32),
            pltpu.VMEM((ZCH,), jnp.float32),
            pltpu.VMEM_SHARED((NP,), jnp.float32),
            pltpu.SemaphoreType.DMA,
        ],
    )
    def hist(dst_hbm, out_hbm, didx, ones_v, zb, acc, sem):
        c = lax.axis_index("c")
        s = lax.axis_index("s")
        t = c * NS + s

        def fill_z(i, carry):
            zb[pl.ds(i * 16, 16)] = jnp.zeros((16,), jnp.float32)
            return carry
        lax.fori_loop(0, ZCH // 16, fill_z, 0)

        def fill_o(i, carry):
            ones_v[pl.ds(i * 16, 16)] = jnp.full((16,), 1.0, jnp.float32)
            return carry
        lax.fori_loop(0, 128 // 16, fill_o, 0)

        pltpu.sync_copy(zb, acc.at[pl.ds(s * ZCH, ZCH)])
        plsc.subcore_barrier()

        pltpu.sync_copy(dst_hbm.at[pl.ds(t * CH, CH)], didx)

        def fire(j, carry):
            pltpu.async_copy(ones_v, acc.at[didx.at[j]], sem, add=True)
            return carry
        lax.fori_loop(0, CH, fire, 0)

        def drain(j, carry):
            pltpu.make_async_copy(ones_v, acc.at[didx.at[0]], sem).wait()
            return carry
        lax.fori_loop(0, CH, drain, 0)

        plsc.subcore_barrier()
        pltpu.sync_copy(acc.at[pl.ds(s * ZCH, ZCH)], zb)
        pltpu.sync_copy(zb, out_hbm.at[pl.ds(c * NP + s * ZCH, ZCH)])

    return hist


def _make_segsum(NR, R, H, A):
    RPT = NR // NS
    PH = 8
    TOT = R // (NS * PH)
    B = TOT - A
    assert 0 < A < TOT

    @functools.partial(
        pl.kernel,
        out_type=jax.ShapeDtypeStruct((NC, NR, H), jnp.float32),
        mesh=_mesh(),
        scratch_types=[
            pltpu.VMEM((PH, 128), jnp.int32),
            pltpu.VMEM((PH, 128), jnp.int32),
            pltpu.VMEM((128, H), jnp.float32),
            pltpu.VMEM((128, H), jnp.float32),
            pltpu.VMEM_SHARED((NR, H), jnp.float32),
            pltpu.SemaphoreType.DMA,
            pltpu.SemaphoreType.DMA,
            pltpu.SemaphoreType.DMA,
            pltpu.SemaphoreType.DMA,
        ],
    )
    def segsum(g_hbm, src_hbm, dst_hbm, out_hbm,
               sidx, didx, buf0, buf1, acc, gs0, gs1, ss0, ss1):
        c = lax.axis_index("c")
        s = lax.axis_index("s")

        def zrow(i, carry):
            r = i // (H // 16)
            k = i % (H // 16)
            buf0[r, pl.ds(k * 16, 16)] = jnp.zeros((16,), jnp.float32)
            return carry
        lax.fori_loop(0, 128 * (H // 16), zrow, 0)

        base = s * RPT
        off = 0
        rem = RPT
        while rem > 0:
            sz = min(128, rem)
            pltpu.sync_copy(buf0.at[pl.ds(0, sz)],
                            acc.at[pl.ds(base + off, sz)])
            off += sz
            rem -= sz
        plsc.subcore_barrier()

        def run_phases(nph, row0):
            def phase_body(ph, carry):
                row = row0 + ph * PH
                pltpu.sync_copy(src_hbm.at[pl.ds(row, PH)], sidx)
                pltpu.sync_copy(dst_hbm.at[pl.ds(row, PH)], didx)
                pltpu.async_copy(g_hbm.at[sidx.at[0]], buf0, gs0)

                def pair(p, carry2):
                    j0 = 2 * p
                    j1 = j0 + 1
                    pltpu.make_async_copy(
                        g_hbm.at[sidx.at[j0]], buf0, gs0).wait()
                    pltpu.async_copy(buf0, acc.at[didx.at[j0]], ss0,
                                     add=True)

                    @pl.when(p > 0)
                    def _():
                        pltpu.make_async_copy(
                            buf1, acc.at[didx.at[j1]], ss1).wait()

                    pltpu.async_copy(g_hbm.at[sidx.at[j1]], buf1, gs1)
                    pltpu.make_async_copy(
                        g_hbm.at[sidx.at[j1]], buf1, gs1).wait()
                    pltpu.async_copy(buf1, acc.at[didx.at[j1]], ss1,
                                     add=True)
                    pltpu.make_async_copy(
                        buf0, acc.at[didx.at[j0]], ss0).wait()

                    @pl.when(j1 + 1 < PH)
                    def _():
                        pltpu.async_copy(g_hbm.at[sidx.at[j1 + 1]], buf0, gs0)

                    return carry2
                lax.fori_loop(0, PH // 2, pair, 0)
                pltpu.make_async_copy(
                    buf1, acc.at[didx.at[PH - 1]], ss1).wait()
                return carry
            lax.fori_loop(0, nph, phase_body, 0)

        @pl.when(c == 0)
        def _():
            run_phases(A, s * (A * PH))

        @pl.when(c == 1)
        def _():
            run_phases(B, NS * A * PH + s * (B * PH))

        plsc.subcore_barrier()
        off = 0
        rem = RPT
        while rem > 0:
            sz = min(128, rem)
            pltpu.sync_copy(acc.at[pl.ds(base + off, sz)],
                            buf0.at[pl.ds(0, sz)])
            pltpu.sync_copy(buf0.at[pl.ds(0, sz)],
                            out_hbm.at[c, pl.ds(base + off, sz)])
            off += sz
            rem -= sz

    return segsum


def _dinv_col(degp, N):
    NP = degp.shape[1]
    rows = NP // 128

    def body(d_ref, o_ref):
        d = d_ref[0] + d_ref[1] + 1.0
        o_ref[...] = lax.rsqrt(d)

    dinv2d = pl.pallas_call(
        body,
        out_shape=jax.ShapeDtypeStruct((rows, 128), jnp.float32),
    )(degp.reshape(NC, rows, 128))
    return dinv2d.reshape(NP)[:N, None]


def _tc_in(x, W, dinv_col, B=400):
    N, D = x.shape
    H = W.shape[1]

    def body(x_ref, w_ref, dv_ref, o_ref):
        o_ref[...] = dv_ref[...] * jnp.dot(
            x_ref[...], w_ref[...], preferred_element_type=jnp.float32)

    return pl.pallas_call(
        body, grid=(N // B,),
        in_specs=[pl.BlockSpec((B, D), lambda i: (i, 0)),
                  pl.BlockSpec((D, H), lambda i: (0, 0)),
                  pl.BlockSpec((B, 1), lambda i: (i, 0))],
        out_specs=pl.BlockSpec((B, H), lambda i: (i, 0)),
        out_shape=jax.ShapeDtypeStruct((N, H), jnp.float32),
    )(x, W, dinv_col)


def _tc_mid(s1p, g1, dinv_col, b1, gamma1, beta1, W2, B=400):
    N, H = g1.shape
    NR = s1p.shape[1]
    bnscale = 1.0 / math.sqrt(1.0 + 1e-5)

    def body(s_ref, g_ref, dv_ref, b_ref, ga_ref, be_ref, w_ref,
             x1_ref, g2_ref):
        dv = dv_ref[...]
        pre = dv * (s_ref[0] + s_ref[1] + g_ref[...]) + b_ref[...]
        x1 = jnp.maximum(pre * (ga_ref[...] * bnscale) + be_ref[...], 0.0)
        x1_ref[...] = x1
        g2_ref[...] = dv * jnp.dot(
            x1, w_ref[...], preferred_element_type=jnp.float32)

    return pl.pallas_call(
        body, grid=(N // B,),
        in_specs=[pl.BlockSpec((NC, B, H), lambda i: (0, i, 0)),
                  pl.BlockSpec((B, H), lambda i: (i, 0)),
                  pl.BlockSpec((B, 1), lambda i: (i, 0)),
                  pl.BlockSpec((1, H), lambda i: (0, 0)),
                  pl.BlockSpec((1, H), lambda i: (0, 0)),
                  pl.BlockSpec((1, H), lambda i: (0, 0)),
                  pl.BlockSpec((H, H), lambda i: (0, 0))],
        out_specs=[pl.BlockSpec((B, H), lambda i: (i, 0)),
                   pl.BlockSpec((B, H), lambda i: (i, 0))],
        out_shape=[jax.ShapeDtypeStruct((N, H), jnp.float32),
                   jax.ShapeDtypeStruct((N, H), jnp.float32)],
    )(s1p, g1, dinv_col, b1, gamma1, beta1, W2)


def _tc_out(s2p, g2, x1, dinv_col, b2, Wf, bf, B=400):
    N, H = g2.shape
    C = Wf.shape[1]

    def body(s_ref, g_ref, x1_ref, dv_ref, b_ref, wf_ref, bf_ref, o_ref):
        x2 = dv_ref[...] * (s_ref[0] + s_ref[1] + g_ref[...]) + b_ref[...]
        xj = jnp.maximum(x1_ref[...], x2)
        o = jnp.dot(xj, wf_ref[...],
                    preferred_element_type=jnp.float32) + bf_ref[...]
        m = jnp.max(o, axis=1, keepdims=True)
        e = jnp.exp(o - m)
        o_ref[...] = (o - m) - jnp.log(jnp.sum(e, axis=1, keepdims=True))

    return pl.pallas_call(
        body, grid=(N // B,),
        in_specs=[pl.BlockSpec((NC, B, H), lambda i: (0, i, 0)),
                  pl.BlockSpec((B, H), lambda i: (i, 0)),
                  pl.BlockSpec((B, H), lambda i: (i, 0)),
                  pl.BlockSpec((B, 1), lambda i: (i, 0)),
                  pl.BlockSpec((1, H), lambda i: (0, 0)),
                  pl.BlockSpec((H, C), lambda i: (0, 0)),
                  pl.BlockSpec((1, C), lambda i: (0, 0))],
        out_specs=pl.BlockSpec((B, C), lambda i: (i, 0)),
        out_shape=jax.ShapeDtypeStruct((N, C), jnp.float32),
    )(s2p, g2, x1, dinv_col, b2, Wf, bf)


def kernel(x, edge_index, W1, b1, gamma1, beta1, W2, b2, Wf, bf):
    N, D = x.shape
    H = W1.shape[1]
    E = edge_index.shape[1]

    EP = ((E + NW * 2048 - 1) // (NW * 2048)) * (NW * 2048)
    CH = EP // (NW * 128)
    pad = EP - E
    src = edge_index[0].astype(jnp.int32)
    dst = edge_index[1].astype(jnp.int32)
    src2d = jnp.concatenate([src, jnp.zeros((pad,), jnp.int32)]).reshape(-1, 128)
    dst2d = jnp.concatenate([dst, jnp.full((pad,), N, jnp.int32)]).reshape(-1, 128)

    NP = ((N + 1 + 127) // 128) * 128
    NR = NP

    degp = _make_hist(NP, CH)(dst2d).reshape(NC, NP)
    dinv_col = _dinv_col(degp, N)

    R = EP // 128
    TOT = R // (NS * 8)
    A = max(1, min(TOT - 1, (TOT * 18 + 10) // 20))
    g1 = _tc_in(x, W1, dinv_col)
    s1p = _make_segsum(NR, R, H, A)(g1, src2d, dst2d)
    x1, g2 = _tc_mid(s1p, g1, dinv_col,
                     b1[None, :], gamma1[None, :], beta1[None, :], W2)
    s2p = _make_segsum(NR, R, H, A)(g2, src2d, dst2d)
    return _tc_out(s2p, g2, x1, dinv_col, b2[None, :], Wf, bf[None, :])


# speedup vs baseline: 1.1071x; 1.0049x over previous
import functools
import math

import jax
import jax.numpy as jnp
from jax import lax
from jax.experimental import pallas as pl
from jax.experimental.pallas import tpu as pltpu
from jax.experimental.pallas import tpu_sc as plsc

NC, NS, LANES = 2, 16, 16
NW = NC * NS


def _mesh():
    return plsc.VectorSubcoreMesh(
        core_axis_name="c", subcore_axis_name="s",
        num_cores=NC, num_subcores=NS)


def _make_hist(NP, CH):
    ZCH = NP // NS

    @functools.partial(
        pl.kernel,
        out_type=jax.ShapeDtypeStruct((NC * NP,), jnp.float32),
        mesh=_mesh(),
        scratch_types=[
            pltpu.VMEM((CH, 128), jnp.int32),
            pltpu.VMEM((128,), jnp.float32),
            pltpu.VMEM((ZCH,), jnp.float32),
            pltpu.VMEM_SHARED((NP,), jnp.float32),
            pltpu.SemaphoreType.DMA,
        ],
    )
    def hist(dst_hbm, out_hbm, didx, ones_v, zb, acc, sem):
        c = lax.axis_index("c")
        s = lax.axis_index("s")
        t = c * NS + s

        def fill_z(i, carry):
            zb[pl.ds(i * 16, 16)] = jnp.zeros((16,), jnp.float32)
            return carry
        lax.fori_loop(0, ZCH // 16, fill_z, 0)

        def fill_o(i, carry):
            ones_v[pl.ds(i * 16, 16)] = jnp.full((16,), 1.0, jnp.float32)
            return carry
        lax.fori_loop(0, 128 // 16, fill_o, 0)

        pltpu.sync_copy(zb, acc.at[pl.ds(s * ZCH, ZCH)])
        plsc.subcore_barrier()

        pltpu.sync_copy(dst_hbm.at[pl.ds(t * CH, CH)], didx)

        def fire(j, carry):
            pltpu.async_copy(ones_v, acc.at[didx.at[j]], sem, add=True)
            return carry
        lax.fori_loop(0, CH, fire, 0)

        def drain(j, carry):
            pltpu.make_async_copy(ones_v, acc.at[didx.at[0]], sem).wait()
            return carry
        lax.fori_loop(0, CH, drain, 0)

        plsc.subcore_barrier()
        pltpu.sync_copy(acc.at[pl.ds(s * ZCH, ZCH)], zb)
        pltpu.sync_copy(zb, out_hbm.at[pl.ds(c * NP + s * ZCH, ZCH)])

    return hist


def _make_segsum(NR, R, H, A):
    RPT = NR // NS
    PH = 8
    TOT = R // (NS * PH)
    B = TOT - A
    assert 0 < A < TOT

    @functools.partial(
        pl.kernel,
        out_type=jax.ShapeDtypeStruct((NC, NR, H), jnp.float32),
        mesh=_mesh(),
        scratch_types=[
            pltpu.VMEM((PH, 128), jnp.int32),
            pltpu.VMEM((PH, 128), jnp.int32),
            pltpu.VMEM((128, H), jnp.float32),
            pltpu.VMEM((128, H), jnp.float32),
            pltpu.VMEM_SHARED((NR, H), jnp.float32),
            pltpu.SemaphoreType.DMA,
            pltpu.SemaphoreType.DMA,
            pltpu.SemaphoreType.DMA,
            pltpu.SemaphoreType.DMA,
        ],
    )
    def segsum(g_hbm, src_hbm, dst_hbm, out_hbm,
               sidx, didx, buf0, buf1, acc, gs0, gs1, ss0, ss1):
        c = lax.axis_index("c")
        s = lax.axis_index("s")

        def zrow(i, carry):
            r = i // (H // 16)
            k = i % (H // 16)
            buf0[r, pl.ds(k * 16, 16)] = jnp.zeros((16,), jnp.float32)
            return carry
        lax.fori_loop(0, 128 * (H // 16), zrow, 0)

        base = s * RPT
        off = 0
        rem = RPT
        while rem > 0:
            sz = min(128, rem)
            pltpu.sync_copy(buf0.at[pl.ds(0, sz)],
                            acc.at[pl.ds(base + off, sz)])
            off += sz
            rem -= sz
        plsc.subcore_barrier()

        def run_phases(nph, row0):
            def phase_body(ph, carry):
                row = row0 + ph * PH
                pltpu.async_copy(src_hbm.at[pl.ds(row, PH)], sidx, ss0)
                pltpu.async_copy(dst_hbm.at[pl.ds(row, PH)], didx, ss1)
                pltpu.make_async_copy(
                    src_hbm.at[pl.ds(row, PH)], sidx, ss0).wait()
                pltpu.async_copy(g_hbm.at[sidx.at[0]], buf0, gs0)
                pltpu.make_async_copy(
                    dst_hbm.at[pl.ds(row, PH)], didx, ss1).wait()

                def pair(p, carry2):
                    j0 = 2 * p
                    j1 = j0 + 1
                    pltpu.make_async_copy(
                        g_hbm.at[sidx.at[j0]], buf0, gs0).wait()
                    pltpu.async_copy(buf0, acc.at[didx.at[j0]], ss0,
                                     add=True)

                    @pl.when(p > 0)
                    def _():
                        pltpu.make_async_copy(
                            buf1, acc.at[didx.at[j1]], ss1).wait()

                    pltpu.async_copy(g_hbm.at[sidx.at[j1]], buf1, gs1)
                    pltpu.make_async_copy(
                        g_hbm.at[sidx.at[j1]], buf1, gs1).wait()
                    pltpu.async_copy(buf1, acc.at[didx.at[j1]], ss1,
                                     add=True)
                    pltpu.make_async_copy(
                        buf0, acc.at[didx.at[j0]], ss0).wait()

                    @pl.when(j1 + 1 < PH)
                    def _():
                        pltpu.async_copy(g_hbm.at[sidx.at[j1 + 1]], buf0, gs0)

                    return carry2
                lax.fori_loop(0, PH // 2, pair, 0)
                pltpu.make_async_copy(
                    buf1, acc.at[didx.at[PH - 1]], ss1).wait()
                return carry
            lax.fori_loop(0, nph, phase_body, 0)

        @pl.when(c == 0)
        def _():
            run_phases(A, s * (A * PH))

        @pl.when(c == 1)
        def _():
            run_phases(B, NS * A * PH + s * (B * PH))

        plsc.subcore_barrier()
        szs = []
        rem = RPT
        while rem > 0:
            szs.append(min(128, rem))
            rem -= szs[-1]
        offs = [sum(szs[:k]) for k in range(len(szs))]
        for k, sz in enumerate(szs):
            b = buf0 if k % 2 == 0 else buf1
            sem = gs0 if k % 2 == 0 else gs1
            if k >= 2:
                pltpu.make_async_copy(
                    b.at[pl.ds(0, szs[k - 2])],
                    out_hbm.at[c, pl.ds(base + offs[k - 2], szs[k - 2])],
                    sem).wait()
            pltpu.sync_copy(acc.at[pl.ds(base + offs[k], sz)],
                            b.at[pl.ds(0, sz)])
            pltpu.async_copy(b.at[pl.ds(0, sz)],
                             out_hbm.at[c, pl.ds(base + offs[k], sz)], sem)
        for k in range(max(0, len(szs) - 2), len(szs)):
            b = buf0 if k % 2 == 0 else buf1
            sem = gs0 if k % 2 == 0 else gs1
            pltpu.make_async_copy(
                b.at[pl.ds(0, szs[k])],
                out_hbm.at[c, pl.ds(base + offs[k], szs[k])], sem).wait()

    return segsum


def _dinv_col(degp, N):
    NP = degp.shape[1]
    rows = NP // 128

    def body(d_ref, o_ref):
        d = d_ref[0] + d_ref[1] + 1.0
        o_ref[...] = lax.rsqrt(d)

    dinv2d = pl.pallas_call(
        body,
        out_shape=jax.ShapeDtypeStruct((rows, 128), jnp.float32),
    )(degp.reshape(NC, rows, 128))
    return dinv2d.reshape(NP)[:N, None]


def _tc_in(x, W, dinv_col, B=400):
    N, D = x.shape
    H = W.shape[1]

    def body(x_ref, w_ref, dv_ref, o_ref):
        o_ref[...] = dv_ref[...] * jnp.dot(
            x_ref[...], w_ref[...], preferred_element_type=jnp.float32)

    return pl.pallas_call(
        body, grid=(N // B,),
        in_specs=[pl.BlockSpec((B, D), lambda i: (i, 0)),
                  pl.BlockSpec((D, H), lambda i: (0, 0)),
                  pl.BlockSpec((B, 1), lambda i: (i, 0))],
        out_specs=pl.BlockSpec((B, H), lambda i: (i, 0)),
        out_shape=jax.ShapeDtypeStruct((N, H), jnp.float32),
    )(x, W, dinv_col)


def _tc_mid(s1p, g1, dinv_col, b1, gamma1, beta1, W2, B=400):
    N, H = g1.shape
    NR = s1p.shape[1]
    bnscale = 1.0 / math.sqrt(1.0 + 1e-5)

    def body(s_ref, g_ref, dv_ref, b_ref, ga_ref, be_ref, w_ref,
             x1_ref, g2_ref):
        dv = dv_ref[...]
        pre = dv * (s_ref[0] + s_ref[1] + g_ref[...]) + b_ref[...]
        x1 = jnp.maximum(pre * (ga_ref[...] * bnscale) + be_ref[...], 0.0)
        x1_ref[...] = x1
        g2_ref[...] = dv * jnp.dot(
            x1, w_ref[...], preferred_element_type=jnp.float32)

    return pl.pallas_call(
        body, grid=(N // B,),
        in_specs=[pl.BlockSpec((NC, B, H), lambda i: (0, i, 0)),
                  pl.BlockSpec((B, H), lambda i: (i, 0)),
                  pl.BlockSpec((B, 1), lambda i: (i, 0)),
                  pl.BlockSpec((1, H), lambda i: (0, 0)),
                  pl.BlockSpec((1, H), lambda i: (0, 0)),
                  pl.BlockSpec((1, H), lambda i: (0, 0)),
                  pl.BlockSpec((H, H), lambda i: (0, 0))],
        out_specs=[pl.BlockSpec((B, H), lambda i: (i, 0)),
                   pl.BlockSpec((B, H), lambda i: (i, 0))],
        out_shape=[jax.ShapeDtypeStruct((N, H), jnp.float32),
                   jax.ShapeDtypeStruct((N, H), jnp.float32)],
    )(s1p, g1, dinv_col, b1, gamma1, beta1, W2)


def _tc_out(s2p, g2, x1, dinv_col, b2, Wf, bf, B=400):
    N, H = g2.shape
    C = Wf.shape[1]

    def body(s_ref, g_ref, x1_ref, dv_ref, b_ref, wf_ref, bf_ref, o_ref):
        x2 = dv_ref[...] * (s_ref[0] + s_ref[1] + g_ref[...]) + b_ref[...]
        xj = jnp.maximum(x1_ref[...], x2)
        o = jnp.dot(xj, wf_ref[...],
                    preferred_element_type=jnp.float32) + bf_ref[...]
        m = jnp.max(o, axis=1, keepdims=True)
        e = jnp.exp(o - m)
        o_ref[...] = (o - m) - jnp.log(jnp.sum(e, axis=1, keepdims=True))

    return pl.pallas_call(
        body, grid=(N // B,),
        in_specs=[pl.BlockSpec((NC, B, H), lambda i: (0, i, 0)),
                  pl.BlockSpec((B, H), lambda i: (i, 0)),
                  pl.BlockSpec((B, H), lambda i: (i, 0)),
                  pl.BlockSpec((B, 1), lambda i: (i, 0)),
                  pl.BlockSpec((1, H), lambda i: (0, 0)),
                  pl.BlockSpec((H, C), lambda i: (0, 0)),
                  pl.BlockSpec((1, C), lambda i: (0, 0))],
        out_specs=pl.BlockSpec((B, C), lambda i: (i, 0)),
        out_shape=jax.ShapeDtypeStruct((N, C), jnp.float32),
    )(s2p, g2, x1, dinv_col, b2, Wf, bf)


def kernel(x, edge_index, W1, b1, gamma1, beta1, W2, b2, Wf, bf):
    N, D = x.shape
    H = W1.shape[1]
    E = edge_index.shape[1]

    EP = ((E + NW * 2048 - 1) // (NW * 2048)) * (NW * 2048)
    CH = EP // (NW * 128)
    pad = EP - E
    src = edge_index[0].astype(jnp.int32)
    dst = edge_index[1].astype(jnp.int32)
    src2d = jnp.concatenate([src, jnp.zeros((pad,), jnp.int32)]).reshape(-1, 128)
    dst2d = jnp.concatenate([dst, jnp.full((pad,), N, jnp.int32)]).reshape(-1, 128)

    NP = ((N + 1 + 127) // 128) * 128
    NR = NP

    degp = _make_hist(NP, CH)(dst2d).reshape(NC, NP)
    dinv_col = _dinv_col(degp, N)

    R = EP // 128
    TOT = R // (NS * 8)
    A = max(1, min(TOT - 1, (TOT * 18 + 10) // 20))
    g1 = _tc_in(x, W1, dinv_col)
    s1p = _make_segsum(NR, R, H, A)(g1, src2d, dst2d)
    x1, g2 = _tc_mid(s1p, g1, dinv_col,
                     b1[None, :], gamma1[None, :], beta1[None, :], W2)
    s2p = _make_segsum(NR, R, H, A)(g2, src2d, dst2d)
    return _tc_out(s2p, g2, x1, dinv_col, b2[None, :], Wf, bf[None, :])
